# baseline (speedup 1.0000x reference)
import functools

import jax
import jax.numpy as jnp
from jax.experimental import pallas as pl
from jax.experimental.pallas import tpu as pltpu

_BN = 3072


def _mm_body(feat_ref, mem_ref, params_ref, out_ref, *, squared):
    t = params_ref[0, 0]
    inv_t = 1.0 / t
    scale = inv_t * inv_t if squared else inv_t
    dims = (((1,), (1,)), ((), ()))
    out_ref[...] = jax.lax.dot_general(
        mem_ref[...], feat_ref[...], dims,
        preferred_element_type=jnp.float32) * scale


def _sim_body(feat_ref, tfeat_ref, sim_ref):
    sim_ref[...] = jnp.sum(feat_ref[...] * tfeat_ref[...], axis=-1,
                           keepdims=True)


def _mm(feat, memory, p2d, *, squared):
    B, D = feat.shape
    N = memory.shape[0]
    return pl.pallas_call(
        functools.partial(_mm_body, squared=squared),
        grid=(pl.cdiv(N, _BN),),
        in_specs=[
            pl.BlockSpec((B, D), lambda j: (0, 0)),
            pl.BlockSpec((_BN, D), lambda j: (j, 0)),
            pl.BlockSpec((1, 2), lambda j: (0, 0)),
        ],
        out_specs=pl.BlockSpec((_BN, B), lambda j: (j, 0)),
        out_shape=jax.ShapeDtypeStruct((N, B), jnp.float32),
        compiler_params=pltpu.CompilerParams(
            dimension_semantics=("parallel",),
        ),
    )(feat, memory, p2d)


def kernel(image_features, transformed_image_features, indices, memory, params):
    del indices
    B, D = image_features.shape
    p2d = params.reshape(1, 2)
    out_f = _mm(image_features, memory, p2d, squared=False)
    out_t = _mm(transformed_image_features, memory, p2d, squared=True)
    sim = pl.pallas_call(
        _sim_body,
        out_shape=jax.ShapeDtypeStruct((B, 1), jnp.float32),
    )(image_features, transformed_image_features)
    return (out_t.T, out_f.T, sim)


# speedup vs baseline: 1.0693x; 1.0693x over previous
import functools

import jax
import jax.numpy as jnp
from jax.experimental import pallas as pl
from jax.experimental.pallas import tpu as pltpu

_BN = 2048
_NBUF = 3


def _body(feat_ref, tfeat_ref, mem_ref, params_ref,
          out_t_hbm, out_f_hbm, sim_ref,
          buf_t, buf_f, sems, *, B, N):
    j = pl.program_id(0)
    nsteps = pl.num_programs(0)
    last = nsteps - 1
    slot = jax.lax.rem(j, _NBUF)
    tail = N - last * _BN

    t = params_ref[0, 0]
    inv_t = 1.0 / t
    f = feat_ref[...]
    tf = tfeat_ref[...]
    m = mem_ref[...]
    dims = (((1,), (1,)), ((), ()))

    @pl.when(j >= _NBUF)
    def _():
        for o, buf in ((0, buf_t), (1, buf_f)):
            pltpu.make_async_copy(
                buf.at[slot], out_t_hbm.at[pl.ds(0, _BN), :],
                sems.at[slot, o],
            ).wait()

    buf_f[slot] = jax.lax.dot_general(
        m, f, dims, preferred_element_type=jnp.float32) * inv_t
    buf_t[slot] = jax.lax.dot_general(
        m, tf, dims, preferred_element_type=jnp.float32) * (inv_t * inv_t)

    col = j * _BN

    @pl.when(j < last)
    def _():
        for o, (buf, hbm) in enumerate(((buf_t, out_t_hbm), (buf_f, out_f_hbm))):
            pltpu.make_async_copy(
                buf.at[slot], hbm.at[pl.ds(col, _BN), :], sems.at[slot, o],
            ).start(priority=o)

    @pl.when(j == last)
    def _():
        for o, (buf, hbm) in enumerate(((buf_t, out_t_hbm), (buf_f, out_f_hbm))):
            pltpu.make_async_copy(
                buf.at[slot, pl.ds(0, tail), :],
                hbm.at[pl.ds(col, tail), :],
                sems.at[slot, o],
            ).start(priority=o)

    @pl.when(j == 0)
    def _():
        sim_ref[...] = jnp.sum(f * tf, axis=-1, keepdims=True)

    @pl.when(j == last)
    def _():
        for k in range(1, _NBUF):
            sl = jax.lax.rem(j - k + _NBUF, _NBUF)

            @pl.when(j - k >= 0)
            def _():
                for o, buf in ((0, buf_t), (1, buf_f)):
                    pltpu.make_async_copy(
                        buf.at[sl], out_t_hbm.at[pl.ds(0, _BN), :],
                        sems.at[sl, o],
                    ).wait()
        for o, buf in ((0, buf_t), (1, buf_f)):
            pltpu.make_async_copy(
                buf.at[slot, pl.ds(0, tail), :],
                out_t_hbm.at[pl.ds(0, tail), :],
                sems.at[slot, o],
            ).wait()


def kernel(image_features, transformed_image_features, indices, memory, params):
    del indices
    B, D = image_features.shape
    N = memory.shape[0]
    grid = (pl.cdiv(N, _BN),)
    p2d = params.reshape(1, 2)
    out_t, out_f, sim = pl.pallas_call(
        functools.partial(_body, B=B, N=N),
        grid=grid,
        in_specs=[
            pl.BlockSpec((B, D), lambda j: (0, 0)),
            pl.BlockSpec((B, D), lambda j: (0, 0)),
            pl.BlockSpec((_BN, D), lambda j: (j, 0)),
            pl.BlockSpec((1, 2), lambda j: (0, 0)),
        ],
        out_specs=[
            pl.BlockSpec(memory_space=pl.ANY),
            pl.BlockSpec(memory_space=pl.ANY),
            pl.BlockSpec((B, 1), lambda j: (0, 0)),
        ],
        out_shape=[
            jax.ShapeDtypeStruct((N, B), jnp.float32),
            jax.ShapeDtypeStruct((N, B), jnp.float32),
            jax.ShapeDtypeStruct((B, 1), jnp.float32),
        ],
        scratch_shapes=[
            pltpu.VMEM((_NBUF, _BN, B), jnp.float32),
            pltpu.VMEM((_NBUF, _BN, B), jnp.float32),
            pltpu.SemaphoreType.DMA((_NBUF, 2)),
        ],
        compiler_params=pltpu.CompilerParams(
            dimension_semantics=("arbitrary",),
        ),
    )(image_features, transformed_image_features, memory, p2d)
    return (out_t.T, out_f.T, sim)
